# baseline (speedup 1.0000x reference)
import functools

import jax
import jax.numpy as jnp
from jax import lax
from jax.experimental import pallas as pl
from jax.experimental.pallas import tpu as pltpu
from jax.experimental.pallas import tpu_sc as plsc

N, V, DIM = 4096, 200, 64
NC = 2
C_PER_W = 2
BV = 8
SPLIT = 96


def _sc_gather(wt_flat, eidx):
    mesh = plsc.VectorSubcoreMesh(core_axis_name="c", subcore_axis_name="s")

    @functools.partial(
        pl.kernel,
        mesh=mesh,
        out_type=jax.ShapeDtypeStruct((DIM * V,), jnp.float32),
        scratch_types=(
            [pltpu.VMEM((SPLIT,), jnp.int32) for _ in range(C_PER_W)]
            + [pltpu.VMEM((V - SPLIT,), jnp.int32) for _ in range(C_PER_W)]
            + [pltpu.VMEM((SPLIT,), jnp.float32) for _ in range(C_PER_W)]
            + [pltpu.VMEM((V - SPLIT,), jnp.float32) for _ in range(C_PER_W)]
            + [pltpu.SemaphoreType.DMA]
        ),
    )
    def gather_kernel(wt_hbm, eidx_hbm, out_hbm, *scratch):
        idx_a = scratch[0:C_PER_W]
        idx_b = scratch[C_PER_W:2 * C_PER_W]
        row_a = scratch[2 * C_PER_W:3 * C_PER_W]
        row_b = scratch[3 * C_PER_W:4 * C_PER_W]
        sem = scratch[4 * C_PER_W]
        wid = lax.axis_index("s") * NC + lax.axis_index("c")
        for k in range(C_PER_W):
            base = (wid * C_PER_W + k) * V
            pltpu.sync_copy(eidx_hbm.at[pl.ds(base, SPLIT)], idx_a[k])
            pltpu.sync_copy(eidx_hbm.at[pl.ds(base + SPLIT, V - SPLIT)],
                            idx_b[k])
        copies = []
        for k in range(C_PER_W):
            copies.append(pltpu.async_copy(wt_hbm.at[idx_a[k]], row_a[k], sem))
            copies.append(pltpu.async_copy(wt_hbm.at[idx_b[k]], row_b[k], sem))
        for c in copies:
            c.wait()
        for k in range(C_PER_W):
            base = (wid * C_PER_W + k) * V
            pltpu.sync_copy(row_a[k], out_hbm.at[pl.ds(base, SPLIT)])
            pltpu.sync_copy(row_b[k], out_hbm.at[pl.ds(base + SPLIT,
                                                       V - SPLIT)])

    return gather_kernel(wt_flat, eidx)


def _add_body(x_ref, e_ref, o_ref):
    o_ref[...] = x_ref[...] + e_ref[...][:, :, None]


def _tc_add(xt, e2):
    return pl.pallas_call(
        _add_body,
        grid=(V // BV,),
        in_specs=[
            pl.BlockSpec((BV, DIM, N), lambda i: (i, 0, 0)),
            pl.BlockSpec((BV, DIM), lambda i: (i, 0)),
        ],
        out_specs=pl.BlockSpec((BV, DIM, N), lambda i: (i, 0, 0)),
        out_shape=jax.ShapeDtypeStruct((V, DIM, N), jnp.float32),
    )(xt, e2)


@jax.jit
def kernel(x, node_types, W):
    rows = W.shape[0]
    idx = node_types.reshape(V)
    eidx = (idx[None, :]
            + rows * jnp.arange(DIM, dtype=jnp.int32)[:, None]).reshape(-1)
    wt_flat = jnp.transpose(W).reshape(DIM * rows)
    e_t = _sc_gather(wt_flat, eidx).reshape(DIM, V)
    embeds = jnp.transpose(e_t)
    xt = jnp.transpose(x, (1, 2, 0))
    out_t = _tc_add(xt, embeds)
    return jnp.transpose(out_t, (2, 0, 1))
